# baseline (speedup 1.0000x reference)
import functools

import jax
import jax.numpy as jnp
from jax import lax
from jax.experimental import pallas as pl
from jax.experimental.pallas import tpu as pltpu
from jax.experimental.pallas import tpu_sc as plsc

NC = 2
NS = 16
L = 16

CHUNK = 128


def _sc_scatter(n_nodes, d, nchunk):
    rows_per_tile = n_nodes // NS
    ngroups = d // L
    blk = -(-rows_per_tile // 8) * 8

    mesh = plsc.VectorSubcoreMesh(core_axis_name="c", subcore_axis_name="s")

    @functools.partial(
        pl.kernel,
        out_type=jax.ShapeDtypeStruct((NC, n_nodes, d), jnp.float32),
        mesh=mesh,
        compiler_params=pltpu.CompilerParams(use_tc_tiling_on_sc=False),
        scratch_types=dict(
            ibuf=pltpu.VMEM((3, CHUNK), jnp.int32),
            rows=pltpu.VMEM((CHUNK, d), jnp.float32),
            rows_pk=pltpu.VMEM((CHUNK, d // 2), jnp.int32),
            semg0=pltpu.SemaphoreType.DMA,
            acc=pltpu.VMEM_SHARED((n_nodes, d), jnp.float32),
        ),
    )
    def sc_kernel(x_hbm, idx_hbm, out_hbm, ibuf, rows, rows_pk, semg0, acc):
        c = lax.axis_index("c")
        s = lax.axis_index("s")
        wid = c * NS + s

        def _zero_row(i, _):
            for g in range(ngroups):
                rows[i, pl.ds(g * L, L)] = jnp.zeros((L,), jnp.float32)
            return 0

        lax.fori_loop(0, CHUNK, _zero_row, 0)
        a = jnp.minimum((s * rows_per_tile) // 8 * 8, n_nodes - blk)
        off = 0
        for h in [CHUNK] * (blk // CHUNK) + ([blk % CHUNK] if blk % CHUNK else []):
            pltpu.sync_copy(rows.at[pl.ds(0, h)], acc.at[pl.ds(a + off, h)])
            off += h
        plsc.subcore_barrier()

        def _chunk(j, _):
            pltpu.sync_copy(idx_hbm.at[wid, j], ibuf)
            pltpu.async_copy(x_hbm.at[ibuf.at[0]], rows_pk, semg0).wait()

            def _scale16(t, _):
                wv = lax.bitcast_convert_type(
                    ibuf[2, pl.ds(t * L, L)], jnp.float32)
                for ee in range(L):
                    wb = jnp.broadcast_to(wv[ee], (L,))
                    e = t * L + ee
                    for g in range(ngroups // 2):
                        v = rows_pk[e, pl.ds(g * L, L)]
                        va = lax.bitcast_convert_type(v << 16, jnp.float32)
                        vb = lax.bitcast_convert_type(
                            v & jnp.int32(-65536), jnp.float32)
                        rows[e, pl.ds(g * 2 * L, L)] = va * wb
                        rows[e, pl.ds(g * 2 * L + L, L)] = vb * wb
                return 0

            lax.fori_loop(0, CHUNK // L, _scale16, 0)

            pltpu.sync_copy(rows, acc.at[ibuf.at[1]], add=True)
            return 0

        lax.fori_loop(0, nchunk, _chunk, 0)

        plsc.subcore_barrier()
        pltpu.sync_copy(acc.at[pl.ds(a, blk)],
                        out_hbm.at[c].at[pl.ds(a, blk)])

    return sc_kernel


def _mm_body(a_ref, w_ref, o_ref):
    s = a_ref[0] + a_ref[1]
    o_ref[...] = jnp.dot(s, w_ref[...], preferred_element_type=jnp.float32)


def kernel(x, edge_index, edge_weight, W):
    n, d = x.shape
    e = edge_weight.shape[0]
    nw = NC * NS

    per_w = -(-e // (nw * CHUNK)) * CHUNK
    e_pad = per_w * nw
    pad = e_pad - e
    nchunk = per_w // CHUNK
    src = jnp.pad(edge_index[0], (0, pad)).reshape(nw, nchunk, 1, CHUNK)
    dst = jnp.pad(edge_index[1], (0, pad)).reshape(nw, nchunk, 1, CHUNK)
    w_i = lax.bitcast_convert_type(jnp.pad(edge_weight, (0, pad)),
                                   jnp.int32).reshape(nw, nchunk, 1, CHUNK)
    idx = jnp.concatenate([src, dst, w_i], axis=2)

    x_bf = (x.reshape(n, d // 32, 2, 16).transpose(0, 1, 3, 2)
            .reshape(n, d // 2, 2).astype(jnp.bfloat16))
    x_pk = lax.bitcast_convert_type(x_bf, jnp.int32)

    partial = _sc_scatter(n, d, nchunk)(x_pk, idx)

    rows_blk = 1000
    out = pl.pallas_call(
        _mm_body,
        grid=(n // rows_blk,),
        in_specs=[
            pl.BlockSpec((NC, rows_blk, d), lambda i: (0, i, 0)),
            pl.BlockSpec((d, d), lambda i: (0, 0)),
        ],
        out_specs=pl.BlockSpec((rows_blk, d), lambda i: (i, 0)),
        out_shape=jax.ShapeDtypeStruct((n, d), jnp.float32),
    )(partial, W)
    return out


# speedup vs baseline: 1.0981x; 1.0981x over previous
import functools

import jax
import jax.numpy as jnp
from jax import lax
from jax.experimental import pallas as pl
from jax.experimental.pallas import tpu as pltpu
from jax.experimental.pallas import tpu_sc as plsc

NC = 2
NS = 16
L = 16

CHUNK = 128


def _sc_scatter(n_nodes, d, nchunk):
    rows_per_tile = n_nodes // NS
    ngroups = d // L
    blk = -(-rows_per_tile // 8) * 8

    mesh = plsc.VectorSubcoreMesh(core_axis_name="c", subcore_axis_name="s")

    @functools.partial(
        pl.kernel,
        out_type=jax.ShapeDtypeStruct((NC, n_nodes, d), jnp.float32),
        mesh=mesh,
        scratch_types=dict(
            ibuf=pltpu.VMEM((3, CHUNK), jnp.int32),
            rows=pltpu.VMEM((CHUNK, d), jnp.float32),
            semg0=pltpu.SemaphoreType.DMA,
            acc=pltpu.VMEM_SHARED((n_nodes, d), jnp.float32),
        ),
    )
    def sc_kernel(x_hbm, idx_hbm, out_hbm, ibuf, rows, semg0, acc):
        c = lax.axis_index("c")
        s = lax.axis_index("s")
        wid = c * NS + s

        def _zero_row(i, _):
            for g in range(ngroups):
                rows[i, pl.ds(g * L, L)] = jnp.zeros((L,), jnp.float32)
            return 0

        lax.fori_loop(0, CHUNK, _zero_row, 0)
        a = jnp.minimum((s * rows_per_tile) // 8 * 8, n_nodes - blk)
        off = 0
        for h in [CHUNK] * (blk // CHUNK) + ([blk % CHUNK] if blk % CHUNK else []):
            pltpu.sync_copy(rows.at[pl.ds(0, h)], acc.at[pl.ds(a + off, h)])
            off += h
        plsc.subcore_barrier()

        def _chunk(j, _):
            pltpu.sync_copy(idx_hbm.at[wid, j], ibuf)
            pltpu.async_copy(x_hbm.at[ibuf.at[0]], rows, semg0).wait()

            def _scale16(t, _):
                wv = lax.bitcast_convert_type(
                    ibuf[2, pl.ds(t * L, L)], jnp.float32)
                for ee in range(L):
                    wb = jnp.broadcast_to(wv[ee], (L,))
                    e = t * L + ee
                    for g in range(ngroups):
                        rows[e, pl.ds(g * L, L)] = rows[e, pl.ds(g * L, L)] * wb
                return 0

            lax.fori_loop(0, CHUNK // L, _scale16, 0)

            pltpu.sync_copy(rows, acc.at[ibuf.at[1]], add=True)
            return 0

        lax.fori_loop(0, nchunk, _chunk, 0)

        plsc.subcore_barrier()
        pltpu.sync_copy(acc.at[pl.ds(a, blk)],
                        out_hbm.at[c].at[pl.ds(a, blk)])

    return sc_kernel


def _mm_body(a_ref, w_ref, o_ref):
    s = a_ref[0] + a_ref[1]
    o_ref[...] = jnp.dot(s, w_ref[...], preferred_element_type=jnp.float32)


def kernel(x, edge_index, edge_weight, W):
    n, d = x.shape
    e = edge_weight.shape[0]
    nw = NC * NS

    per_w = -(-e // (nw * CHUNK)) * CHUNK
    e_pad = per_w * nw
    pad = e_pad - e
    nchunk = per_w // CHUNK
    src = jnp.pad(edge_index[0], (0, pad)).reshape(nw, nchunk, 1, CHUNK)
    dst = jnp.pad(edge_index[1], (0, pad)).reshape(nw, nchunk, 1, CHUNK)
    w_i = lax.bitcast_convert_type(jnp.pad(edge_weight, (0, pad)),
                                   jnp.int32).reshape(nw, nchunk, 1, CHUNK)
    idx = jnp.concatenate([src, dst, w_i], axis=2)

    partial = _sc_scatter(n, d, nchunk)(x, idx)

    rows_blk = 1000
    out = pl.pallas_call(
        _mm_body,
        grid=(n // rows_blk,),
        in_specs=[
            pl.BlockSpec((NC, rows_blk, d), lambda i: (0, i, 0)),
            pl.BlockSpec((d, d), lambda i: (0, 0)),
        ],
        out_specs=pl.BlockSpec((rows_blk, d), lambda i: (i, 0)),
        out_shape=jax.ShapeDtypeStruct((n, d), jnp.float32),
    )(partial, W)
    return out
